# baseline (speedup 1.0000x reference)
import functools

import jax
import jax.numpy as jnp
from jax import lax
from jax.experimental import pallas as pl
from jax.experimental.pallas import tpu as pltpu
from jax.experimental.pallas import tpu_sc as plsc

B_S = 1024
S_L = 200
H = 128
MAX_LEN = 512
N = B_S * S_L
NC, NS, L = 2, 16, 16
NW = NC * NS
RPW = B_S // NW
TPW = N // NW
C = 64
S = 16
NCHUNK = TPW // C
COLS = H // L
NB = 4
DIST = 3
PROWS = MAX_LEN // NS
ROW_OFFS = [k * L for k in range(S_L // L)] + [S_L - L]

_mesh = plsc.VectorSubcoreMesh(core_axis_name="c", subcore_axis_name="s")


@functools.partial(
    pl.kernel,
    out_type=[
        jax.ShapeDtypeStruct((N, H), jnp.float32),
        jax.ShapeDtypeStruct((B_S, S_L), jnp.int32),
    ],
    mesh=_mesh,
    scratch_types=[
        pltpu.VMEM((RPW, S_L), jnp.int32),
        pltpu.VMEM((RPW, S_L), jnp.int32),
        pltpu.VMEM((RPW, S_L), jnp.int32),
        pltpu.VMEM((TPW,), jnp.int32),
        pltpu.VMEM((TPW,), jnp.int32),
        pltpu.VMEM((NB, C, H), jnp.float32),
        pltpu.VMEM((NB, C - S, H), jnp.float32),
        pltpu.VMEM_SHARED((MAX_LEN, H), jnp.float32),
    ] + [pltpu.SemaphoreType.DMA] * (4 * NB + 3),
)
def _emb(inp_hbm, pos_hbm, w_hbm, p_hbm, out_hbm, mask_hbm,
         tok2, pos2, msk2, tokf, posf, wrows, prows, p_sh, *sems):
    semw = sems[:NB]
    semp = sems[NB:2 * NB]
    semo = sems[2 * NB:3 * NB]
    sema = sems[3 * NB:4 * NB]
    semt, semq, semm = sems[4 * NB:]
    wid = lax.axis_index("s") * NC + lax.axis_index("c")
    sid = lax.axis_index("s")
    base = wid * TPW
    row0 = wid * RPW

    prow0 = sid * PROWS
    pltpu.async_copy(p_hbm.at[pl.ds(prow0, PROWS)],
                     p_sh.at[pl.ds(prow0, PROWS)], semm)
    pltpu.async_copy(inp_hbm.at[pl.ds(row0, RPW)], tok2, semt)
    pltpu.async_copy(pos_hbm.at[pl.ds(row0, RPW)], pos2, semq)
    pltpu.make_async_copy(inp_hbm.at[pl.ds(row0, RPW)], tok2, semt).wait()
    pltpu.make_async_copy(pos_hbm.at[pl.ds(row0, RPW)], pos2, semq).wait()

    ones = jnp.ones((L,), jnp.int32)

    def repack_row(r, _):
        for c in ROW_OFFS:
            t = tok2[r, pl.ds(c, L)]
            p = pos2[r, pl.ds(c, L)]
            nonpad = jnp.minimum(jnp.abs(t), ones)
            tokf[pl.ds(r * S_L + c, L)] = t
            posf[pl.ds(r * S_L + c, L)] = p * nonpad
            msk2[r, pl.ds(c, L)] = ones - nonpad
        return 0

    lax.fori_loop(0, RPW, repack_row, 0)
    pltpu.async_copy(msk2, mask_hbm.at[pl.ds(row0, RPW)], semm)
    pltpu.make_async_copy(p_hbm.at[pl.ds(0, PROWS)],
                          p_sh.at[pl.ds(0, PROWS)], semm).wait()
    plsc.subcore_barrier()

    def issue_gathers(j, b):
        pltpu.async_copy(w_hbm.at[tokf.at[pl.ds(j * C, C)]],
                         wrows.at[b], semw[b])
        pltpu.async_copy(p_sh.at[posf.at[pl.ds(j * C + S, C - S)]],
                         prows.at[b], semp[b])

    def issue_stream_add(j, b):
        pltpu.async_copy(p_sh.at[posf.at[pl.ds(j * C, S)]],
                         wrows.at[b, pl.ds(0, S)], sema[b], add=True)

    def wait_w(b):
        pltpu.make_async_copy(w_hbm.at[pl.ds(0, C)],
                              wrows.at[b], semw[b]).wait()

    def wait_p(b):
        pltpu.make_async_copy(w_hbm.at[pl.ds(0, C - S)],
                              prows.at[b], semp[b]).wait()

    def wait_sadd(b):
        pltpu.make_async_copy(w_hbm.at[pl.ds(0, S)],
                              wrows.at[b, pl.ds(0, S)], sema[b]).wait()

    def wait_store(b):
        pltpu.make_async_copy(w_hbm.at[pl.ds(0, C)],
                              wrows.at[b], semo[b]).wait()

    for j in range(DIST):
        issue_gathers(j, j)
    wait_w(0)
    issue_stream_add(0, 0)

    def outer(j0, _):
        for b in range(NB):
            j = j0 * NB + b
            b1 = (b + 1) % NB

            @pl.when(j + 1 < NCHUNK)
            def _start_next_add():
                wait_w(b1)
                issue_stream_add(j + 1, b1)

            wait_p(b)

            def add_row(r, _):
                for c in range(COLS):
                    sl = pl.ds(c * L, L)
                    plsc.addupdate(wrows.at[b, S + r, sl], prows[b, r, sl])
                return 0

            lax.fori_loop(0, C - S, add_row, 0)
            wait_sadd(b)
            pltpu.async_copy(wrows.at[b],
                             out_hbm.at[pl.ds(base + j * C, C)], semo[b])
            jn = j + DIST

            @pl.when(j >= NB - DIST)
            def _drain_store():
                wait_store((b + DIST) % NB)

            @pl.when(jn < NCHUNK)
            def _issue_next():
                issue_gathers(jn, (b + DIST) % NB)
        return 0

    lax.fori_loop(0, NCHUNK // NB, outer, 0)
    wait_store((NCHUNK - 1) % NB)
    pltpu.make_async_copy(msk2, mask_hbm.at[pl.ds(row0, RPW)], semm).wait()


def kernel(input, positional, W, P):
    inp = input.astype(jnp.int32)
    pos = positional.astype(jnp.int32)
    out, mask = _emb(inp, pos, W, P)
    return out.reshape(B_S, S_L, H), mask.astype(bool)


# speedup vs baseline: 1.1787x; 1.1787x over previous
import functools

import jax
import jax.numpy as jnp
from jax import lax
from jax.experimental import pallas as pl
from jax.experimental.pallas import tpu as pltpu
from jax.experimental.pallas import tpu_sc as plsc

B_S = 1024
S_L = 200
H = 128
MAX_LEN = 512
N = B_S * S_L
NC, NS, L = 2, 16, 16
NW = NC * NS
RPW = B_S // NW
TPW = N // NW
C = 64
NCHUNK = TPW // C
COLS = H // L
NB = 5
DIST = 4
UNROLL = 2
PROWS = MAX_LEN // NS
ROW_OFFS = [k * L for k in range(S_L // L)] + [S_L - L]

_mesh = plsc.VectorSubcoreMesh(core_axis_name="c", subcore_axis_name="s")


@functools.partial(
    pl.kernel,
    out_type=[
        jax.ShapeDtypeStruct((N, H), jnp.float32),
        jax.ShapeDtypeStruct((B_S, S_L), jnp.int32),
    ],
    mesh=_mesh,
    scratch_types=[
        pltpu.VMEM((RPW, S_L), jnp.int32),
        pltpu.VMEM((RPW, S_L), jnp.int32),
        pltpu.VMEM((RPW, S_L), jnp.int32),
        pltpu.VMEM((TPW,), jnp.int32),
        pltpu.VMEM((TPW,), jnp.int32),
        pltpu.VMEM((NB, C, H), jnp.float32),
        pltpu.VMEM((NB, C, H), jnp.float32),
        pltpu.VMEM_SHARED((MAX_LEN, H), jnp.float32),
    ] + [pltpu.SemaphoreType.DMA] * (3 * NB + 3),
)
def _emb(inp_hbm, pos_hbm, w_hbm, p_hbm, out_hbm, mask_hbm,
         tok2, pos2, msk2, tokf, posf, wrows, prows, p_sh, *sems):
    semw = sems[:NB]
    semp = sems[NB:2 * NB]
    semo = sems[2 * NB:3 * NB]
    semt, semq, semm = sems[3 * NB:]
    wid = lax.axis_index("s") * NC + lax.axis_index("c")
    sid = lax.axis_index("s")
    base = wid * TPW
    row0 = wid * RPW

    prow0 = sid * PROWS
    pltpu.async_copy(p_hbm.at[pl.ds(prow0, PROWS)],
                     p_sh.at[pl.ds(prow0, PROWS)], semm)
    pltpu.async_copy(inp_hbm.at[pl.ds(row0, RPW)], tok2, semt)
    pltpu.async_copy(pos_hbm.at[pl.ds(row0, RPW)], pos2, semq)
    pltpu.make_async_copy(inp_hbm.at[pl.ds(row0, RPW)], tok2, semt).wait()
    pltpu.make_async_copy(pos_hbm.at[pl.ds(row0, RPW)], pos2, semq).wait()

    ones = jnp.ones((L,), jnp.int32)

    def repack_row(r, _):
        for c in ROW_OFFS:
            t = tok2[r, pl.ds(c, L)]
            p = pos2[r, pl.ds(c, L)]
            nonpad = jnp.minimum(jnp.abs(t), ones)
            tokf[pl.ds(r * S_L + c, L)] = t
            posf[pl.ds(r * S_L + c, L)] = p * nonpad
            msk2[r, pl.ds(c, L)] = ones - nonpad
        return 0

    lax.fori_loop(0, RPW, repack_row, 0)
    pltpu.async_copy(msk2, mask_hbm.at[pl.ds(row0, RPW)], semm)
    pltpu.make_async_copy(p_hbm.at[pl.ds(0, PROWS)],
                          p_sh.at[pl.ds(0, PROWS)], semm).wait()
    plsc.subcore_barrier()

    def issue_gathers(j, b):
        pltpu.async_copy(w_hbm.at[tokf.at[pl.ds(j * C, C)]],
                         wrows.at[b], semw[b])
        pltpu.async_copy(p_sh.at[posf.at[pl.ds(j * C, C)]],
                         prows.at[b], semp[b])

    def wait_slot(sem, b):
        pltpu.make_async_copy(w_hbm.at[pl.ds(0, C)], wrows.at[b], sem).wait()

    for j in range(DIST):
        issue_gathers(j, j)

    def outer(j0, _):
        for b in range(NB):
            j = j0 * NB + b
            wait_slot(semw[b], b)
            wait_slot(semp[b], b)

            def add_rows(i, _):
                for u in range(UNROLL):
                    for c in range(COLS):
                        sl = pl.ds(c * L, L)
                        plsc.addupdate(wrows.at[b, i * UNROLL + u, sl],
                                       prows[b, i * UNROLL + u, sl])
                return 0

            lax.fori_loop(0, C // UNROLL, add_rows, 0)
            pltpu.async_copy(wrows.at[b],
                             out_hbm.at[pl.ds(base + j * C, C)], semo[b])
            jn = j + DIST

            @pl.when(j >= NB - DIST)
            def _drain_store():
                wait_slot(semo[(b + DIST) % NB], b)

            @pl.when(jn < NCHUNK)
            def _issue_next():
                issue_gathers(jn, (b + DIST) % NB)
        return 0

    lax.fori_loop(0, NCHUNK // NB, outer, 0)
    wait_slot(semo[(NCHUNK - 1) % NB], 0)
    pltpu.make_async_copy(msk2, mask_hbm.at[pl.ds(row0, RPW)], semm).wait()


def kernel(input, positional, W, P):
    inp = input.astype(jnp.int32)
    pos = positional.astype(jnp.int32)
    out, mask = _emb(inp, pos, W, P)
    return out.reshape(B_S, S_L, H), mask.astype(bool)
